# baseline (speedup 1.0000x reference)
import functools
import math

import jax
import jax.numpy as jnp
from jax import lax
from jax.experimental import pallas as pl
from jax.experimental.pallas import tpu as pltpu
from jax.experimental.pallas import tpu_sc as plsc

SEQ = 200
D = 64
BATCH = 4096
N = BATCH * SEQ

NC = 2
NS = 16
NW = NC * NS

SEQ_PER_W = BATCH // NW
CH = 4
ROWS = CH * SEQ
NCHUNK = SEQ_PER_W // CH
GSUB = 100
NG = ROWS // GSUB

SCALE = math.sqrt(float(D))


def _positional_encoding():
    position = jnp.arange(0, SEQ, 1, dtype=jnp.float32).reshape(-1, 1)
    multiplication = jnp.exp(
        -jnp.arange(0, D * 2, 2, dtype=jnp.float32) * math.log(10000.0) / D)
    excessive = position * multiplication
    pe = jnp.zeros((SEQ, D), dtype=jnp.float32)
    pe = pe.at[:, 0::2].set(jnp.sin(excessive[:, 0::2]))
    pe = pe.at[:, 1::2].set(jnp.cos(excessive[:, 1::2]))
    return pe


def _body(table_hbm, stream_hbm, pe_hbm, out_hbm, idx_v, rows_v, pe_v, sem):
    wid = lax.axis_index("s") * NC + lax.axis_index("c")
    pltpu.sync_copy(pe_hbm, pe_v)

    def chunk(g, carry):
        base = (wid * SEQ_PER_W + g * CH) * SEQ
        pltpu.sync_copy(
            stream_hbm.at[pl.ds(pl.multiple_of(base // GSUB, 8), NG)], idx_v)
        for j in range(NG):
            pltpu.async_copy(
                table_hbm.at[idx_v.at[j]],
                rows_v.at[pl.ds(j * GSUB, GSUB)],
                sem,
            )
        for j in range(NG):
            pltpu.make_async_copy(
                table_hbm.at[idx_v.at[j]],
                rows_v.at[pl.ds(j * GSUB, GSUB)],
                sem,
            ).wait()

        def rowloop(r, c2):
            for ch in range(CH):
                rr = ch * SEQ + r
                for c in range(D // 16):
                    sl = pl.ds(c * 16, 16)
                    rows_v[rr, sl] = rows_v[rr, sl] * SCALE + pe_v[r, sl]
            return c2

        lax.fori_loop(0, SEQ, rowloop, 0)
        pltpu.sync_copy(rows_v, out_hbm.at[pl.ds(base, ROWS)])
        return carry

    lax.fori_loop(0, NCHUNK, chunk, 0)


@jax.jit
def _run(stream2, table, pe):
    mesh = plsc.VectorSubcoreMesh(core_axis_name="c", subcore_axis_name="s",
                                  num_cores=NC, num_subcores=NS)
    f = pl.kernel(
        _body,
        out_type=jax.ShapeDtypeStruct((N, D), jnp.float32),
        mesh=mesh,
        scratch_types=[
            pltpu.VMEM((NG, GSUB), jnp.int32),
            pltpu.VMEM((ROWS, D), jnp.float32),
            pltpu.VMEM((SEQ, D), jnp.float32),
            pltpu.SemaphoreType.DMA,
        ],
        compiler_params=pltpu.CompilerParams(use_tc_tiling_on_sc=False),
    )
    return f(table, stream2, pe)


def kernel(stream, table):
    pe = _positional_encoding()
    stream2 = stream.reshape(N // GSUB, GSUB)
    out = _run(stream2, table, pe)
    return out.reshape(BATCH, SEQ, D)
